# Initial kernel scaffold; baseline (speedup 1.0000x reference)
import jax
import jax.numpy as jnp
from jax.experimental import pallas as pl


def kernel(x1, edge_index1, edge_attr1, batch1, x2, edge_index2, edge_attr2, t_value, params):
    raise NotImplementedError("write your pallas kernel here")


# speedup vs baseline: 3.3996x; 3.3996x over previous
import functools

import jax
import jax.numpy as jnp
from jax import lax
from jax.experimental import pallas as pl
from jax.experimental.pallas import tpu as pltpu
from jax.experimental.pallas import tpu_sc as plsc

H = 64
N = 10000
E = 320000
B = 128
L = 3

NC, NS = 2, 16
NW = NC * NS
EPW = E // NW

_PREC = lax.Precision.HIGHEST


def _dot(a, b):
    return jnp.dot(a, b, precision=_PREC, preferred_element_type=jnp.float32)


def _enc_body(t_ref, batch_ref, w0, b0, w1, b1, w2, b2, out_ref):
    t = t_ref[...]
    h = jnp.maximum(t * w0[...][0:1, :] + b0[...], 0.0)
    h = jnp.maximum(_dot(h, w1[...]) + b1[...], 0.0)
    enc = _dot(h, w2[...]) + b2[...]
    iota = lax.broadcasted_iota(jnp.int32, (N, B), 1)
    oh = (batch_ref[...] == iota).astype(jnp.float32)
    out_ref[...] = _dot(oh, enc)


def _encode(t_value, batch2d, p):
    return pl.pallas_call(
        _enc_body,
        out_shape=jax.ShapeDtypeStruct((N, H), jnp.float32),
    )(t_value, batch2d,
      p['te_w0'], p['te_b0'].reshape(1, H),
      p['te_w1'], p['te_b1'].reshape(1, H),
      p['te_w2'], p['te_b2'].reshape(1, H))


_EEB = 4000


def _ee_body(s_ref, w0, b0, w1, b1, w2, b2, out_ref):
    s = s_ref[...]
    h = jnp.maximum(s * w0[...][0:1, :] + b0[...], 0.0)
    h = jnp.maximum(_dot(h, w1[...]) + b1[...], 0.0)
    out_ref[...] = _dot(h, w2[...]) + b2[...]


def _edge_mlp(s_all, p):
    n = s_all.shape[0]
    grid = n // _EEB
    wspec = pl.BlockSpec((1, H), lambda i: (0, 0))
    return pl.pallas_call(
        _ee_body,
        grid=(grid,),
        in_specs=[pl.BlockSpec((_EEB, 1), lambda i: (i, 0)),
                  wspec, wspec, pl.BlockSpec((H, H), lambda i: (0, 0)), wspec,
                  pl.BlockSpec((H, H), lambda i: (0, 0)), wspec],
        out_specs=pl.BlockSpec((_EEB, H), lambda i: (i, 0)),
        out_shape=jax.ShapeDtypeStruct((n, H), jnp.float32),
    )(s_all,
      p['ee_w0'], p['ee_b0'].reshape(1, H),
      p['ee_w1'], p['ee_b1'].reshape(1, H),
      p['ee_w2'], p['ee_b2'].reshape(1, H))


def _xlxr_body(h_ref, wl, wr, xl_ref, xr_ref):
    h = h_ref[...]
    xl_ref[...] = _dot(h, wl[...])
    xr_ref[...] = _dot(h, wr[...])


def _xlxr(h, Wl, Wr):
    return pl.pallas_call(
        _xlxr_body,
        out_shape=(jax.ShapeDtypeStruct((N, H), jnp.float32),
                   jax.ShapeDtypeStruct((N, H), jnp.float32)),
    )(h, Wl, Wr)


_LB = 4000


def _logits_body(gs_ref, gd_ref, ee_ref, we, att, logit_ref, gmax_ref):
    i = pl.program_id(0)
    m = gs_ref[...] + gd_ref[...] + _dot(ee_ref[...], we[...])
    m = jnp.where(m >= 0.0, m, 0.2 * m)
    lg = _dot(m, att[...])
    logit_ref[...] = lg
    bm = jnp.max(lg)
    prev = jnp.where(i == 0, -jnp.inf, gmax_ref[0, 0])
    gmax_ref[...] = jnp.broadcast_to(jnp.maximum(prev, bm), (1, 1))


def _logits(gs, gd, ee, We, att):
    grid = E // _LB
    return pl.pallas_call(
        _logits_body,
        grid=(grid,),
        in_specs=[pl.BlockSpec((_LB, H), lambda i: (i, 0)),
                  pl.BlockSpec((_LB, H), lambda i: (i, 0)),
                  pl.BlockSpec((_LB, H), lambda i: (i, 0)),
                  pl.BlockSpec((H, H), lambda i: (0, 0)),
                  pl.BlockSpec((H, 1), lambda i: (0, 0))],
        out_specs=(pl.BlockSpec((_LB, 1), lambda i: (i, 0)),
                   pl.BlockSpec((1, 1), lambda i: (0, 0))),
        out_shape=(jax.ShapeDtypeStruct((E, 1), jnp.float32),
                   jax.ShapeDtypeStruct((1, 1), jnp.float32)),
    )(gs, gd, ee, We, att)


def _aw_body(lg_ref, gmax_ref, gs_ref, a16_ref, w_ref):
    a = jnp.exp(lg_ref[...] - gmax_ref[...])
    w_ref[...] = gs_ref[...] * a
    col = lax.broadcasted_iota(jnp.int32, (_LB, 16), 1)
    a16_ref[...] = jnp.where(col == 0, a, 0.0)


def _aw(logits, gmax, gs):
    grid = E // _LB
    return pl.pallas_call(
        _aw_body,
        grid=(grid,),
        in_specs=[pl.BlockSpec((_LB, 1), lambda i: (i, 0)),
                  pl.BlockSpec((1, 1), lambda i: (0, 0)),
                  pl.BlockSpec((_LB, H), lambda i: (i, 0))],
        out_specs=(pl.BlockSpec((_LB, 16), lambda i: (i, 0)),
                   pl.BlockSpec((_LB, H), lambda i: (i, 0))),
        out_shape=(jax.ShapeDtypeStruct((E, 16), jnp.float32),
                   jax.ShapeDtypeStruct((E, H), jnp.float32)),
    )(logits, gmax, gs)


def _norm_body(acc_ref, den_ref, b_ref, out_ref):
    acc = acc_ref[0] + acc_ref[1]
    den = den_ref[0, :, 0:1] + den_ref[1, :, 0:1]
    out_ref[...] = jnp.maximum(acc / (den + 1e-16) + b_ref[...], 0.0)


def _norm(acc, den, b):
    return pl.pallas_call(
        _norm_body,
        out_shape=jax.ShapeDtypeStruct((N, H), jnp.float32),
    )(acc, den, b)


_DB = 2000


def _dec_body(hs_ref, hd_ref, w0a, w0b, b0, w1, b1, w2, b2, out_ref):
    h = jnp.maximum(
        _dot(hs_ref[...], w0a[...]) + _dot(hd_ref[...], w0b[...]) + b0[...],
        0.0)
    h = jnp.maximum(_dot(h, w1[...]) + b1[...], 0.0)
    out_ref[...] = _dot(h, w2[...]) + b2[...]


def _decode(hs, hd, p):
    grid = E // _DB
    bspec = pl.BlockSpec((1, H), lambda i: (0, 0))
    return pl.pallas_call(
        _dec_body,
        grid=(grid,),
        in_specs=[pl.BlockSpec((_DB, 2 * H), lambda i: (i, 0)),
                  pl.BlockSpec((_DB, 2 * H), lambda i: (i, 0)),
                  pl.BlockSpec((2 * H, H), lambda i: (0, 0)),
                  pl.BlockSpec((2 * H, H), lambda i: (0, 0)),
                  bspec,
                  pl.BlockSpec((H, H), lambda i: (0, 0)), bspec,
                  pl.BlockSpec((H, 1), lambda i: (0, 0)),
                  pl.BlockSpec((1, 1), lambda i: (0, 0))],
        out_specs=pl.BlockSpec((_DB, 1), lambda i: (i, 0)),
        out_shape=jax.ShapeDtypeStruct((E, 1), jnp.float32),
    )(hs, hd,
      p['dec_w0'][:2 * H], p['dec_w0'][2 * H:], p['dec_b0'].reshape(1, H),
      p['dec_w1'], p['dec_b1'].reshape(1, H),
      p['dec_w2'], p['dec_b2'].reshape(1, 1))


_SC_PARAMS = pltpu.CompilerParams(use_tc_tiling_on_sc=False)


def _sc_gather2(table_a, table_b, idx_a, idx_b, D, CH):
    mesh = plsc.VectorSubcoreMesh(core_axis_name="c", subcore_axis_name="s")

    @functools.partial(
        pl.kernel, mesh=mesh,
        out_type=(jax.ShapeDtypeStruct((E, D), jnp.float32),
                  jax.ShapeDtypeStruct((E, D), jnp.float32)),
        scratch_types=[pltpu.VMEM((CH,), jnp.int32),
                       pltpu.VMEM((CH, D), jnp.float32),
                       pltpu.SemaphoreType.DMA],
        compiler_params=_SC_PARAMS,
    )
    def k(ta, tb, ia, ib, oa, ob, idx_v, rows_v, sem):
        wid = lax.axis_index("c") * NS + lax.axis_index("s")
        base = wid * EPW

        @pl.loop(0, EPW, step=CH)
        def _(off):
            pltpu.sync_copy(ia.at[pl.ds(base + off, CH)], idx_v)
            pltpu.async_copy(ta.at[idx_v], rows_v, sem).wait()
            pltpu.sync_copy(rows_v, oa.at[pl.ds(base + off, CH)])

        @pl.loop(0, EPW, step=CH)
        def _(off):
            pltpu.sync_copy(ib.at[pl.ds(base + off, CH)], idx_v)
            pltpu.async_copy(tb.at[idx_v], rows_v, sem).wait()
            pltpu.sync_copy(rows_v, ob.at[pl.ds(base + off, CH)])

    return k(table_a, table_b, idx_a, idx_b)


_SCH = 1000
_NRS = N // NS


def _sc_scatter(w, a16, dst, z64, z16):
    mesh = plsc.VectorSubcoreMesh(core_axis_name="c", subcore_axis_name="s")

    @functools.partial(
        pl.kernel, mesh=mesh,
        out_type=(jax.ShapeDtypeStruct((NC, N, H), jnp.float32),
                  jax.ShapeDtypeStruct((NC, N, 16), jnp.float32)),
        scratch_types=[pltpu.VMEM((_SCH,), jnp.int32),
                       pltpu.VMEM((_SCH, H), jnp.float32),
                       pltpu.VMEM((_SCH, 16), jnp.float32),
                       pltpu.VMEM_SHARED((N, H), jnp.float32),
                       pltpu.VMEM_SHARED((N, 16), jnp.float32),
                       pltpu.SemaphoreType.DMA],
        compiler_params=_SC_PARAMS,
    )
    def k(w_hbm, a_hbm, d_hbm, z64_hbm, z16_hbm, acc_out, den_out,
          idx_v, w_v, a_v, acc_s, den_s, sem):
        c = lax.axis_index("c")
        s = lax.axis_index("s")

        @pl.when(s == 0)
        def _():
            pltpu.sync_copy(z64_hbm, acc_s)
            pltpu.sync_copy(z16_hbm, den_s)

        plsc.subcore_barrier()

        base = (c * NS + s) * EPW

        @pl.loop(0, EPW, step=_SCH)
        def _(off):
            pltpu.sync_copy(d_hbm.at[pl.ds(base + off, _SCH)], idx_v)
            pltpu.sync_copy(w_hbm.at[pl.ds(base + off, _SCH)], w_v)
            pltpu.sync_copy(a_hbm.at[pl.ds(base + off, _SCH)], a_v)
            pltpu.sync_copy(w_v, acc_s.at[idx_v], add=True)
            pltpu.sync_copy(a_v, den_s.at[idx_v], add=True)

        plsc.subcore_barrier()

        rb = s * _NRS
        pltpu.sync_copy(acc_s.at[pl.ds(rb, _NRS)],
                        acc_out.at[c, pl.ds(rb, _NRS)])
        pltpu.sync_copy(den_s.at[pl.ds(rb, _NRS)],
                        den_out.at[c, pl.ds(rb, _NRS)])

    return k(w, a16, dst, z64, z16)


def _conv(h, src, dst, ee, Wl, Wr, We, att, b, z64, z16):
    xl, xr = _xlxr(h, Wl, Wr)
    gs, gd = _sc_gather2(xl, xr, src, dst, H, 1000)
    logits, gmax = _logits(gs, gd, ee, We, att)
    a16, w = _aw(logits, gmax, gs)
    acc, den = _sc_scatter(w, a16, dst, z64, z16)
    return _norm(acc, den, b)


def kernel(x1, edge_index1, edge_attr1, batch1, x2, edge_index2, edge_attr2,
           t_value, params):
    p = params
    src1 = edge_index1[0].astype(jnp.int32)
    dst1 = edge_index1[1].astype(jnp.int32)
    src2 = edge_index2[0].astype(jnp.int32)
    dst2 = edge_index2[1].astype(jnp.int32)
    batch2d = batch1.reshape(N, 1).astype(jnp.int32)

    t_enc = _encode(t_value, batch2d, p)
    s_all = jnp.concatenate([edge_attr1[:, 0:1], edge_attr2], axis=0)
    ee_all = _edge_mlp(s_all, p)
    ee1 = ee_all[:E]
    ee2 = ee_all[E:]

    h1 = jnp.concatenate([t_enc, t_enc], axis=1)
    h2 = h1
    z64 = jnp.zeros((N, H), jnp.float32)
    z16 = jnp.zeros((N, 16), jnp.float32)

    for i in range(L):
        o1 = _conv(h1, src1, dst1, ee1,
                   p['gg_Wl'][i], p['gg_Wr'][i], p['gg_We'][i],
                   p['gg_att'][i].reshape(H, 1), p['gg_b'][i].reshape(1, H),
                   z64, z16)
        o2 = _conv(h2, src2, dst2, ee2,
                   p['gf_Wl'][i], p['gf_Wr'][i], p['gf_We'][i],
                   p['gf_att'][i].reshape(H, 1), p['gf_b'][i].reshape(1, H),
                   z64, z16)
        h1 = jnp.concatenate([o2, o1], axis=1)
        h2 = jnp.concatenate([o1, o2], axis=1)

    hs, hd = _sc_gather2(h1, h1, src1, dst1, 2 * H, 200)
    return _decode(hs, hd, p)


# speedup vs baseline: 3.4621x; 1.0184x over previous
import functools

import jax
import jax.numpy as jnp
from jax import lax
from jax.experimental import pallas as pl
from jax.experimental.pallas import tpu as pltpu
from jax.experimental.pallas import tpu_sc as plsc

H = 64
N = 10000
E = 320000
B = 128
L = 3

NC, NS = 2, 16
NW = NC * NS
EPW = E // NW

_PREC = lax.Precision.HIGHEST


def _dot(a, b):
    return jnp.dot(a, b, precision=_PREC, preferred_element_type=jnp.float32)


def _enc_body(t_ref, batch_ref, w0, b0, w1, b1, w2, b2, out_ref):
    t = t_ref[...]
    h = jnp.maximum(t * w0[...][0:1, :] + b0[...], 0.0)
    h = jnp.maximum(_dot(h, w1[...]) + b1[...], 0.0)
    enc = _dot(h, w2[...]) + b2[...]
    iota = lax.broadcasted_iota(jnp.int32, (N, B), 1)
    oh = (batch_ref[...] == iota).astype(jnp.float32)
    out_ref[...] = _dot(oh, enc)


def _encode(t_value, batch2d, p):
    return pl.pallas_call(
        _enc_body,
        out_shape=jax.ShapeDtypeStruct((N, H), jnp.float32),
    )(t_value, batch2d,
      p['te_w0'], p['te_b0'].reshape(1, H),
      p['te_w1'], p['te_b1'].reshape(1, H),
      p['te_w2'], p['te_b2'].reshape(1, H))


_EEB = 4000


def _ee_body(s_ref, w0, b0, w1, b1, w2, b2, out_ref):
    s = s_ref[...]
    h = jnp.maximum(s * w0[...][0:1, :] + b0[...], 0.0)
    h = jnp.maximum(_dot(h, w1[...]) + b1[...], 0.0)
    out_ref[...] = _dot(h, w2[...]) + b2[...]


def _edge_mlp(s_all, p):
    n = s_all.shape[0]
    grid = n // _EEB
    wspec = pl.BlockSpec((1, H), lambda i: (0, 0))
    return pl.pallas_call(
        _ee_body,
        grid=(grid,),
        in_specs=[pl.BlockSpec((_EEB, 1), lambda i: (i, 0)),
                  wspec, wspec, pl.BlockSpec((H, H), lambda i: (0, 0)), wspec,
                  pl.BlockSpec((H, H), lambda i: (0, 0)), wspec],
        out_specs=pl.BlockSpec((_EEB, H), lambda i: (i, 0)),
        out_shape=jax.ShapeDtypeStruct((n, H), jnp.float32),
    )(s_all,
      p['ee_w0'], p['ee_b0'].reshape(1, H),
      p['ee_w1'], p['ee_b1'].reshape(1, H),
      p['ee_w2'], p['ee_b2'].reshape(1, H))


def _xlxr_body(h_ref, wl, wr, xl_ref, xr_ref):
    h = h_ref[...]
    xl_ref[...] = _dot(h, wl[...])
    xr_ref[...] = _dot(h, wr[...])


def _xlxr(h, Wl, Wr):
    return pl.pallas_call(
        _xlxr_body,
        out_shape=(jax.ShapeDtypeStruct((N, H), jnp.float32),
                   jax.ShapeDtypeStruct((N, H), jnp.float32)),
    )(h, Wl, Wr)


_LB = 4000


def _logits_body(gs_ref, gd_ref, ee_ref, we, att, logit_ref, gmax_ref):
    i = pl.program_id(0)
    m = gs_ref[...] + gd_ref[...] + _dot(ee_ref[...], we[...])
    m = jnp.where(m >= 0.0, m, 0.2 * m)
    lg = _dot(m, att[...])
    logit_ref[...] = lg
    bm = jnp.max(lg)
    prev = jnp.where(i == 0, -jnp.inf, gmax_ref[0, 0])
    gmax_ref[...] = jnp.broadcast_to(jnp.maximum(prev, bm), (1, 1))


def _logits(gs, gd, ee, We, att):
    grid = E // _LB
    return pl.pallas_call(
        _logits_body,
        grid=(grid,),
        in_specs=[pl.BlockSpec((_LB, H), lambda i: (i, 0)),
                  pl.BlockSpec((_LB, H), lambda i: (i, 0)),
                  pl.BlockSpec((_LB, H), lambda i: (i, 0)),
                  pl.BlockSpec((H, H), lambda i: (0, 0)),
                  pl.BlockSpec((H, 1), lambda i: (0, 0))],
        out_specs=(pl.BlockSpec((_LB, 1), lambda i: (i, 0)),
                   pl.BlockSpec((1, 1), lambda i: (0, 0))),
        out_shape=(jax.ShapeDtypeStruct((E, 1), jnp.float32),
                   jax.ShapeDtypeStruct((1, 1), jnp.float32)),
    )(gs, gd, ee, We, att)


def _aw_body(lg_ref, gmax_ref, gs_ref, a16_ref, w_ref):
    a = jnp.exp(lg_ref[...] - gmax_ref[...])
    w_ref[...] = gs_ref[...] * a
    col = lax.broadcasted_iota(jnp.int32, (_LB, 16), 1)
    a16_ref[...] = jnp.where(col == 0, a, 0.0)


def _aw(logits, gmax, gs):
    grid = E // _LB
    return pl.pallas_call(
        _aw_body,
        grid=(grid,),
        in_specs=[pl.BlockSpec((_LB, 1), lambda i: (i, 0)),
                  pl.BlockSpec((1, 1), lambda i: (0, 0)),
                  pl.BlockSpec((_LB, H), lambda i: (i, 0))],
        out_specs=(pl.BlockSpec((_LB, 16), lambda i: (i, 0)),
                   pl.BlockSpec((_LB, H), lambda i: (i, 0))),
        out_shape=(jax.ShapeDtypeStruct((E, 16), jnp.float32),
                   jax.ShapeDtypeStruct((E, H), jnp.float32)),
    )(logits, gmax, gs)


def _norm_body(acc_ref, den_ref, b_ref, out_ref):
    acc = acc_ref[0] + acc_ref[1]
    den = den_ref[0, :, 0:1] + den_ref[1, :, 0:1]
    out_ref[...] = jnp.maximum(acc / (den + 1e-16) + b_ref[...], 0.0)


def _norm(acc, den, b):
    return pl.pallas_call(
        _norm_body,
        out_shape=jax.ShapeDtypeStruct((N, H), jnp.float32),
    )(acc, den, b)


_DB = 2000


def _dec_body(hs_ref, hd_ref, w0a, w0b, b0, w1, b1, w2, b2, out_ref):
    h = jnp.maximum(
        _dot(hs_ref[...], w0a[...]) + _dot(hd_ref[...], w0b[...]) + b0[...],
        0.0)
    h = jnp.maximum(_dot(h, w1[...]) + b1[...], 0.0)
    out_ref[...] = _dot(h, w2[...]) + b2[...]


def _decode(hs, hd, p):
    grid = E // _DB
    bspec = pl.BlockSpec((1, H), lambda i: (0, 0))
    return pl.pallas_call(
        _dec_body,
        grid=(grid,),
        in_specs=[pl.BlockSpec((_DB, 2 * H), lambda i: (i, 0)),
                  pl.BlockSpec((_DB, 2 * H), lambda i: (i, 0)),
                  pl.BlockSpec((2 * H, H), lambda i: (0, 0)),
                  pl.BlockSpec((2 * H, H), lambda i: (0, 0)),
                  bspec,
                  pl.BlockSpec((H, H), lambda i: (0, 0)), bspec,
                  pl.BlockSpec((H, 1), lambda i: (0, 0)),
                  pl.BlockSpec((1, 1), lambda i: (0, 0))],
        out_specs=pl.BlockSpec((_DB, 1), lambda i: (i, 0)),
        out_shape=jax.ShapeDtypeStruct((E, 1), jnp.float32),
    )(hs, hd,
      p['dec_w0'][:2 * H], p['dec_w0'][2 * H:], p['dec_b0'].reshape(1, H),
      p['dec_w1'], p['dec_b1'].reshape(1, H),
      p['dec_w2'], p['dec_b2'].reshape(1, 1))


_SC_PARAMS = pltpu.CompilerParams(use_tc_tiling_on_sc=False)


def _gather_pipe(table, idx_hbm, out_hbm, base, nch, CH, idx_v, rows_v, sems):
    pltpu.sync_copy(idx_hbm.at[pl.ds(base, CH)], idx_v[0])
    for i in range(nch):
        b = i % 2
        pltpu.async_copy(table.at[idx_v[b]], rows_v[b], sems[b])
        if i > 0:
            pb = (i - 1) % 2
            pltpu.make_async_copy(table.at[idx_v[pb]], rows_v[pb],
                                  sems[pb]).wait()
            pltpu.sync_copy(rows_v[pb],
                            out_hbm.at[pl.ds(base + (i - 1) * CH, CH)])
        if i + 1 < nch:
            pltpu.sync_copy(idx_hbm.at[pl.ds(base + (i + 1) * CH, CH)],
                            idx_v[(i + 1) % 2])
    lb = (nch - 1) % 2
    pltpu.make_async_copy(table.at[idx_v[lb]], rows_v[lb], sems[lb]).wait()
    pltpu.sync_copy(rows_v[lb], out_hbm.at[pl.ds(base + (nch - 1) * CH, CH)])


def _sc_gather2(table_a, table_b, idx_a, idx_b, D, CH):
    mesh = plsc.VectorSubcoreMesh(core_axis_name="c", subcore_axis_name="s")
    nch = EPW // CH

    @functools.partial(
        pl.kernel, mesh=mesh,
        out_type=(jax.ShapeDtypeStruct((E, D), jnp.float32),
                  jax.ShapeDtypeStruct((E, D), jnp.float32)),
        scratch_types=[pltpu.VMEM((CH,), jnp.int32),
                       pltpu.VMEM((CH,), jnp.int32),
                       pltpu.VMEM((CH, D), jnp.float32),
                       pltpu.VMEM((CH, D), jnp.float32),
                       pltpu.SemaphoreType.DMA,
                       pltpu.SemaphoreType.DMA],
        compiler_params=_SC_PARAMS,
    )
    def k(ta, tb, ia, ib, oa, ob, i0, i1, r0, r1, s0, s1):
        wid = lax.axis_index("c") * NS + lax.axis_index("s")
        base = wid * EPW
        _gather_pipe(ta, ia, oa, base, nch, CH, [i0, i1], [r0, r1], [s0, s1])
        _gather_pipe(tb, ib, ob, base, nch, CH, [i0, i1], [r0, r1], [s0, s1])

    return k(table_a, table_b, idx_a, idx_b)


_SCH = 400
_NRS = N // NS


def _sc_scatter(w, a16, dst, z64, z16):
    mesh = plsc.VectorSubcoreMesh(core_axis_name="c", subcore_axis_name="s")
    nch = EPW // _SCH

    @functools.partial(
        pl.kernel, mesh=mesh,
        out_type=(jax.ShapeDtypeStruct((NC, N, H), jnp.float32),
                  jax.ShapeDtypeStruct((NC, N, 16), jnp.float32)),
        scratch_types=[pltpu.VMEM((_SCH,), jnp.int32),
                       pltpu.VMEM((_SCH,), jnp.int32),
                       pltpu.VMEM((_SCH, H), jnp.float32),
                       pltpu.VMEM((_SCH, H), jnp.float32),
                       pltpu.VMEM((_SCH, 16), jnp.float32),
                       pltpu.VMEM((_SCH, 16), jnp.float32),
                       pltpu.VMEM_SHARED((N, H), jnp.float32),
                       pltpu.VMEM_SHARED((N, 16), jnp.float32),
                       pltpu.SemaphoreType.DMA,
                       pltpu.SemaphoreType.DMA],
        compiler_params=_SC_PARAMS,
    )
    def k(w_hbm, a_hbm, d_hbm, z64_hbm, z16_hbm, acc_out, den_out,
          i0, i1, w0, w1, a0, a1, acc_s, den_s, s0, s1):
        c = lax.axis_index("c")
        s = lax.axis_index("s")
        rb = s * _NRS

        pltpu.sync_copy(z64_hbm.at[pl.ds(rb, _NRS)],
                        acc_s.at[pl.ds(rb, _NRS)])
        pltpu.sync_copy(z16_hbm.at[pl.ds(rb, _NRS)],
                        den_s.at[pl.ds(rb, _NRS)])
        plsc.subcore_barrier()

        base = (c * NS + s) * EPW
        idx_v, w_v, a_v = [i0, i1], [w0, w1], [a0, a1]
        sems = [s0, s1]

        def load(i):
            b = i % 2
            off = base + i * _SCH
            pltpu.async_copy(d_hbm.at[pl.ds(off, _SCH)], idx_v[b], sems[b])
            pltpu.async_copy(w_hbm.at[pl.ds(off, _SCH)], w_v[b], sems[b])
            pltpu.async_copy(a_hbm.at[pl.ds(off, _SCH)], a_v[b], sems[b])

        def wait_load(i):
            b = i % 2
            off = base + i * _SCH
            pltpu.make_async_copy(d_hbm.at[pl.ds(off, _SCH)], idx_v[b],
                                  sems[b]).wait()
            pltpu.make_async_copy(w_hbm.at[pl.ds(off, _SCH)], w_v[b],
                                  sems[b]).wait()
            pltpu.make_async_copy(a_hbm.at[pl.ds(off, _SCH)], a_v[b],
                                  sems[b]).wait()

        load(0)
        for i in range(nch):
            b = i % 2
            wait_load(i)
            if i + 1 < nch:
                load(i + 1)
            pltpu.sync_copy(w_v[b], acc_s.at[idx_v[b]], add=True)
            pltpu.sync_copy(a_v[b], den_s.at[idx_v[b]], add=True)

        plsc.subcore_barrier()
        pltpu.sync_copy(acc_s.at[pl.ds(rb, _NRS)],
                        acc_out.at[c, pl.ds(rb, _NRS)])
        pltpu.sync_copy(den_s.at[pl.ds(rb, _NRS)],
                        den_out.at[c, pl.ds(rb, _NRS)])

    return k(w, a16, dst, z64, z16)


def _conv(h, src, dst, ee, Wl, Wr, We, att, b, z64, z16):
    xl, xr = _xlxr(h, Wl, Wr)
    gs, gd = _sc_gather2(xl, xr, src, dst, H, 400)
    logits, gmax = _logits(gs, gd, ee, We, att)
    a16, w = _aw(logits, gmax, gs)
    acc, den = _sc_scatter(w, a16, dst, z64, z16)
    return _norm(acc, den, b)


def kernel(x1, edge_index1, edge_attr1, batch1, x2, edge_index2, edge_attr2,
           t_value, params):
    p = params
    src1 = edge_index1[0].astype(jnp.int32)
    dst1 = edge_index1[1].astype(jnp.int32)
    src2 = edge_index2[0].astype(jnp.int32)
    dst2 = edge_index2[1].astype(jnp.int32)
    batch2d = batch1.reshape(N, 1).astype(jnp.int32)

    t_enc = _encode(t_value, batch2d, p)
    s_all = jnp.concatenate([edge_attr1[:, 0:1], edge_attr2], axis=0)
    ee_all = _edge_mlp(s_all, p)
    ee1 = ee_all[:E]
    ee2 = ee_all[E:]

    h1 = jnp.concatenate([t_enc, t_enc], axis=1)
    h2 = h1
    z64 = jnp.zeros((N, H), jnp.float32)
    z16 = jnp.zeros((N, 16), jnp.float32)

    for i in range(L):
        o1 = _conv(h1, src1, dst1, ee1,
                   p['gg_Wl'][i], p['gg_Wr'][i], p['gg_We'][i],
                   p['gg_att'][i].reshape(H, 1), p['gg_b'][i].reshape(1, H),
                   z64, z16)
        o2 = _conv(h2, src2, dst2, ee2,
                   p['gf_Wl'][i], p['gf_Wr'][i], p['gf_We'][i],
                   p['gf_att'][i].reshape(H, 1), p['gf_b'][i].reshape(1, H),
                   z64, z16)
        h1 = jnp.concatenate([o2, o1], axis=1)
        h2 = jnp.concatenate([o1, o2], axis=1)

    hs, hd = _sc_gather2(h1, h1, src1, dst1, 2 * H, 200)
    return _decode(hs, hd, p)


# speedup vs baseline: 3.9317x; 1.1356x over previous
import functools

import jax
import jax.numpy as jnp
from jax import lax
from jax.experimental import pallas as pl
from jax.experimental.pallas import tpu as pltpu
from jax.experimental.pallas import tpu_sc as plsc

H = 64
N = 10000
E = 320000
B = 128
L = 3

NC, NS = 2, 16
NW = NC * NS
EPW = E // NW

_PREC = lax.Precision.HIGHEST


def _dot(a, b):
    return jnp.dot(a, b, precision=_PREC, preferred_element_type=jnp.float32)


def _enc_body(t_ref, batch_ref, w0, b0, w1, b1, w2, b2, out_ref):
    t = t_ref[...]
    h = jnp.maximum(t * w0[...][0:1, :] + b0[...], 0.0)
    h = jnp.maximum(_dot(h, w1[...]) + b1[...], 0.0)
    enc = _dot(h, w2[...]) + b2[...]
    iota = lax.broadcasted_iota(jnp.int32, (N, B), 1)
    oh = (batch_ref[...] == iota).astype(jnp.float32)
    out_ref[...] = _dot(oh, enc)


def _encode(t_value, batch2d, p):
    return pl.pallas_call(
        _enc_body,
        out_shape=jax.ShapeDtypeStruct((N, H), jnp.float32),
    )(t_value, batch2d,
      p['te_w0'], p['te_b0'].reshape(1, H),
      p['te_w1'], p['te_b1'].reshape(1, H),
      p['te_w2'], p['te_b2'].reshape(1, H))


_EEB = 4000


def _ee_body(s_ref, w0, b0, w1, b1, w2, b2, out_ref):
    s = s_ref[...]
    h = jnp.maximum(s * w0[...][0:1, :] + b0[...], 0.0)
    h = jnp.maximum(_dot(h, w1[...]) + b1[...], 0.0)
    out_ref[...] = _dot(h, w2[...]) + b2[...]


def _edge_mlp(s_all, p):
    n = s_all.shape[0]
    grid = n // _EEB
    wspec = pl.BlockSpec((1, H), lambda i: (0, 0))
    return pl.pallas_call(
        _ee_body,
        grid=(grid,),
        in_specs=[pl.BlockSpec((_EEB, 1), lambda i: (i, 0)),
                  wspec, wspec, pl.BlockSpec((H, H), lambda i: (0, 0)), wspec,
                  pl.BlockSpec((H, H), lambda i: (0, 0)), wspec],
        out_specs=pl.BlockSpec((_EEB, H), lambda i: (i, 0)),
        out_shape=jax.ShapeDtypeStruct((n, H), jnp.float32),
    )(s_all,
      p['ee_w0'], p['ee_b0'].reshape(1, H),
      p['ee_w1'], p['ee_b1'].reshape(1, H),
      p['ee_w2'], p['ee_b2'].reshape(1, H))


def _xlxr_body(h_ref, wl, wr, xl_ref, xr_ref):
    h = h_ref[...]
    xl_ref[...] = _dot(h, wl[...])
    xr_ref[...] = _dot(h, wr[...])


def _xlxr(h, Wl, Wr):
    return pl.pallas_call(
        _xlxr_body,
        out_shape=(jax.ShapeDtypeStruct((N, H), jnp.float32),
                   jax.ShapeDtypeStruct((N, H), jnp.float32)),
    )(h, Wl, Wr)


_LB = 4000
_WA = H + 16


def _fused_body(gs_ref, gd_ref, ee_ref, we, att, wa_ref):
    gs = gs_ref[...]
    m = gs + gd_ref[...] + _dot(ee_ref[...], we[...])
    m = jnp.where(m >= 0.0, m, 0.2 * m)
    lg = _dot(m, att[...])
    a = jnp.exp(jnp.minimum(lg, 60.0))
    col = lax.broadcasted_iota(jnp.int32, (_LB, 16), 1)
    a16 = jnp.where(col == 0, a, 0.0)
    wa_ref[...] = jnp.concatenate([gs * a, a16], axis=1)


def _fused(gs, gd, ee, We, att):
    grid = E // _LB
    return pl.pallas_call(
        _fused_body,
        grid=(grid,),
        in_specs=[pl.BlockSpec((_LB, H), lambda i: (i, 0)),
                  pl.BlockSpec((_LB, H), lambda i: (i, 0)),
                  pl.BlockSpec((_LB, H), lambda i: (i, 0)),
                  pl.BlockSpec((H, H), lambda i: (0, 0)),
                  pl.BlockSpec((H, 1), lambda i: (0, 0))],
        out_specs=pl.BlockSpec((_LB, _WA), lambda i: (i, 0)),
        out_shape=jax.ShapeDtypeStruct((E, _WA), jnp.float32),
    )(gs, gd, ee, We, att)


def _norm_body(acc_ref, b_ref, out_ref):
    num = acc_ref[0, :, :H] + acc_ref[1, :, :H]
    den = acc_ref[0, :, H:H + 1] + acc_ref[1, :, H:H + 1]
    out_ref[...] = jnp.maximum(num / (den + 1e-16) + b_ref[...], 0.0)


def _norm(acc, b):
    return pl.pallas_call(
        _norm_body,
        out_shape=jax.ShapeDtypeStruct((N, H), jnp.float32),
    )(acc, b)


_DB = 2000


def _dec_body(hs_ref, hd_ref, w0a, w0b, b0, w1, b1, w2, b2, out_ref):
    h = jnp.maximum(
        _dot(hs_ref[...], w0a[...]) + _dot(hd_ref[...], w0b[...]) + b0[...],
        0.0)
    h = jnp.maximum(_dot(h, w1[...]) + b1[...], 0.0)
    out_ref[...] = _dot(h, w2[...]) + b2[...]


def _decode(hs, hd, p):
    grid = E // _DB
    bspec = pl.BlockSpec((1, H), lambda i: (0, 0))
    return pl.pallas_call(
        _dec_body,
        grid=(grid,),
        in_specs=[pl.BlockSpec((_DB, 2 * H), lambda i: (i, 0)),
                  pl.BlockSpec((_DB, 2 * H), lambda i: (i, 0)),
                  pl.BlockSpec((2 * H, H), lambda i: (0, 0)),
                  pl.BlockSpec((2 * H, H), lambda i: (0, 0)),
                  bspec,
                  pl.BlockSpec((H, H), lambda i: (0, 0)), bspec,
                  pl.BlockSpec((H, 1), lambda i: (0, 0)),
                  pl.BlockSpec((1, 1), lambda i: (0, 0))],
        out_specs=pl.BlockSpec((_DB, 1), lambda i: (i, 0)),
        out_shape=jax.ShapeDtypeStruct((E, 1), jnp.float32),
    )(hs, hd,
      p['dec_w0'][:2 * H], p['dec_w0'][2 * H:], p['dec_b0'].reshape(1, H),
      p['dec_w1'], p['dec_b1'].reshape(1, H),
      p['dec_w2'], p['dec_b2'].reshape(1, 1))


_SC_PARAMS = pltpu.CompilerParams(use_tc_tiling_on_sc=False)


def _gather_pipe(table, idx_hbm, out_hbm, base, nch, CH, idx_v, rows_v, sems):
    pltpu.sync_copy(idx_hbm.at[pl.ds(base, CH)], idx_v[0])
    for i in range(nch):
        b = i % 2
        pltpu.async_copy(table.at[idx_v[b]], rows_v[b], sems[b])
        if i > 0:
            pb = (i - 1) % 2
            pltpu.make_async_copy(table.at[idx_v[pb]], rows_v[pb],
                                  sems[pb]).wait()
            pltpu.sync_copy(rows_v[pb],
                            out_hbm.at[pl.ds(base + (i - 1) * CH, CH)])
        if i + 1 < nch:
            pltpu.sync_copy(idx_hbm.at[pl.ds(base + (i + 1) * CH, CH)],
                            idx_v[(i + 1) % 2])
    lb = (nch - 1) % 2
    pltpu.make_async_copy(table.at[idx_v[lb]], rows_v[lb], sems[lb]).wait()
    pltpu.sync_copy(rows_v[lb], out_hbm.at[pl.ds(base + (nch - 1) * CH, CH)])


def _sc_gather2(table_a, table_b, idx_a, idx_b, D, CH):
    mesh = plsc.VectorSubcoreMesh(core_axis_name="c", subcore_axis_name="s")
    nch = EPW // CH

    @functools.partial(
        pl.kernel, mesh=mesh,
        out_type=(jax.ShapeDtypeStruct((E, D), jnp.float32),
                  jax.ShapeDtypeStruct((E, D), jnp.float32)),
        scratch_types=[pltpu.VMEM((CH,), jnp.int32),
                       pltpu.VMEM((CH,), jnp.int32),
                       pltpu.VMEM((CH, D), jnp.float32),
                       pltpu.VMEM((CH, D), jnp.float32),
                       pltpu.SemaphoreType.DMA,
                       pltpu.SemaphoreType.DMA],
        compiler_params=_SC_PARAMS,
    )
    def k(ta, tb, ia, ib, oa, ob, i0, i1, r0, r1, s0, s1):
        wid = lax.axis_index("c") * NS + lax.axis_index("s")
        base = wid * EPW
        _gather_pipe(ta, ia, oa, base, nch, CH, [i0, i1], [r0, r1], [s0, s1])
        _gather_pipe(tb, ib, ob, base, nch, CH, [i0, i1], [r0, r1], [s0, s1])

    return k(table_a, table_b, idx_a, idx_b)


_SCH = 400
_NRS = N // NS


def _sc_scatter(wa, dst, z80):
    mesh = plsc.VectorSubcoreMesh(core_axis_name="c", subcore_axis_name="s")
    nch = EPW // _SCH

    @functools.partial(
        pl.kernel, mesh=mesh,
        out_type=jax.ShapeDtypeStruct((NC, N, _WA), jnp.float32),
        scratch_types=[pltpu.VMEM((_SCH,), jnp.int32),
                       pltpu.VMEM((_SCH,), jnp.int32),
                       pltpu.VMEM((_SCH, _WA), jnp.float32),
                       pltpu.VMEM((_SCH, _WA), jnp.float32),
                       pltpu.VMEM_SHARED((N, _WA), jnp.float32),
                       pltpu.SemaphoreType.DMA,
                       pltpu.SemaphoreType.DMA],
        compiler_params=_SC_PARAMS,
    )
    def k(wa_hbm, d_hbm, z80_hbm, acc_out, i0, i1, w0, w1, acc_s, s0, s1):
        c = lax.axis_index("c")
        s = lax.axis_index("s")
        rb = s * _NRS

        pltpu.sync_copy(z80_hbm.at[pl.ds(rb, _NRS)],
                        acc_s.at[pl.ds(rb, _NRS)])
        plsc.subcore_barrier()

        base = (c * NS + s) * EPW
        idx_v, w_v = [i0, i1], [w0, w1]
        sems = [s0, s1]

        def load(i):
            b = i % 2
            off = base + i * _SCH
            pltpu.async_copy(d_hbm.at[pl.ds(off, _SCH)], idx_v[b], sems[b])
            pltpu.async_copy(wa_hbm.at[pl.ds(off, _SCH)], w_v[b], sems[b])

        def wait_load(i):
            b = i % 2
            off = base + i * _SCH
            pltpu.make_async_copy(d_hbm.at[pl.ds(off, _SCH)], idx_v[b],
                                  sems[b]).wait()
            pltpu.make_async_copy(wa_hbm.at[pl.ds(off, _SCH)], w_v[b],
                                  sems[b]).wait()

        load(0)
        for i in range(nch):
            b = i % 2
            wait_load(i)
            if i + 1 < nch:
                load(i + 1)
            pltpu.sync_copy(w_v[b], acc_s.at[idx_v[b]], add=True)

        plsc.subcore_barrier()
        pltpu.sync_copy(acc_s.at[pl.ds(rb, _NRS)],
                        acc_out.at[c, pl.ds(rb, _NRS)])

    return k(wa, dst, z80)


def _conv(h, src, dst, ee, Wl, Wr, We, att, b, z80):
    xl, xr = _xlxr(h, Wl, Wr)
    gs, gd = _sc_gather2(xl, xr, src, dst, H, 400)
    wa = _fused(gs, gd, ee, We, att)
    acc = _sc_scatter(wa, dst, z80)
    return _norm(acc, b)


def kernel(x1, edge_index1, edge_attr1, batch1, x2, edge_index2, edge_attr2,
           t_value, params):
    p = params
    src1 = edge_index1[0].astype(jnp.int32)
    dst1 = edge_index1[1].astype(jnp.int32)
    src2 = edge_index2[0].astype(jnp.int32)
    dst2 = edge_index2[1].astype(jnp.int32)
    batch2d = batch1.reshape(N, 1).astype(jnp.int32)

    t_enc = _encode(t_value, batch2d, p)
    s_all = jnp.concatenate([edge_attr1[:, 0:1], edge_attr2], axis=0)
    ee_all = _edge_mlp(s_all, p)
    ee1 = ee_all[:E]
    ee2 = ee_all[E:]

    h1 = jnp.concatenate([t_enc, t_enc], axis=1)
    h2 = h1
    z80 = jnp.zeros((N, _WA), jnp.float32)

    for i in range(L):
        o1 = _conv(h1, src1, dst1, ee1,
                   p['gg_Wl'][i], p['gg_Wr'][i], p['gg_We'][i],
                   p['gg_att'][i].reshape(H, 1), p['gg_b'][i].reshape(1, H),
                   z80)
        o2 = _conv(h2, src2, dst2, ee2,
                   p['gf_Wl'][i], p['gf_Wr'][i], p['gf_We'][i],
                   p['gf_att'][i].reshape(H, 1), p['gf_b'][i].reshape(1, H),
                   z80)
        h1 = jnp.concatenate([o2, o1], axis=1)
        h2 = jnp.concatenate([o1, o2], axis=1)

    hs, hd = _sc_gather2(h1, h1, src1, dst1, 2 * H, 200)
    return _decode(hs, hd, p)


# speedup vs baseline: 6.4094x; 1.6302x over previous
import functools

import jax
import jax.numpy as jnp
from jax import lax
from jax.experimental import pallas as pl
from jax.experimental.pallas import tpu as pltpu
from jax.experimental.pallas import tpu_sc as plsc

H = 64
N = 10000
E = 320000
B = 128
L = 3

NC, NS = 2, 16
NW = NC * NS
EPW = E // NW

_PREC = lax.Precision.DEFAULT


def _dot(a, b):
    return jnp.dot(a, b, precision=_PREC, preferred_element_type=jnp.float32)


def _enc_body(t_ref, batch_ref, w0, b0, w1, b1, w2, b2, out_ref):
    t = t_ref[...]
    h = jnp.maximum(t * w0[...][0:1, :] + b0[...], 0.0)
    h = jnp.maximum(_dot(h, w1[...]) + b1[...], 0.0)
    enc = _dot(h, w2[...]) + b2[...]
    iota = lax.broadcasted_iota(jnp.int32, (N, B), 1)
    oh = (batch_ref[...] == iota).astype(jnp.float32)
    out_ref[...] = _dot(oh, enc)


def _encode(t_value, batch2d, p):
    return pl.pallas_call(
        _enc_body,
        out_shape=jax.ShapeDtypeStruct((N, H), jnp.float32),
    )(t_value, batch2d,
      p['te_w0'], p['te_b0'].reshape(1, H),
      p['te_w1'], p['te_b1'].reshape(1, H),
      p['te_w2'], p['te_b2'].reshape(1, H))


_EEB = 8000


def _ee_body(s_ref, w0, b0, w1, b1, w2, b2, out_ref):
    s = s_ref[...]
    h = jnp.maximum(s * w0[...][0:1, :] + b0[...], 0.0)
    h = jnp.maximum(_dot(h, w1[...]) + b1[...], 0.0)
    out_ref[...] = _dot(h, w2[...]) + b2[...]


def _edge_mlp(s_all, p):
    n = s_all.shape[0]
    grid = n // _EEB
    wspec = pl.BlockSpec((1, H), lambda i: (0, 0))
    return pl.pallas_call(
        _ee_body,
        grid=(grid,),
        in_specs=[pl.BlockSpec((_EEB, 1), lambda i: (i, 0)),
                  wspec, wspec, pl.BlockSpec((H, H), lambda i: (0, 0)), wspec,
                  pl.BlockSpec((H, H), lambda i: (0, 0)), wspec],
        out_specs=pl.BlockSpec((_EEB, H), lambda i: (i, 0)),
        out_shape=jax.ShapeDtypeStruct((n, H), jnp.float32),
    )(s_all,
      p['ee_w0'], p['ee_b0'].reshape(1, H),
      p['ee_w1'], p['ee_b1'].reshape(1, H),
      p['ee_w2'], p['ee_b2'].reshape(1, H))


def _xlxr_body(h_ref, wl, wr, xl_ref, xr_ref):
    h = h_ref[...]
    xl_ref[...] = _dot(h, wl[...])
    xr_ref[...] = _dot(h, wr[...])


def _xlxr(h, Wl, Wr):
    return pl.pallas_call(
        _xlxr_body,
        out_shape=(jax.ShapeDtypeStruct((N, H), jnp.float32),
                   jax.ShapeDtypeStruct((N, H), jnp.float32)),
    )(h, Wl, Wr)


_LB = 8000
_WA = H + 16


def _fused_body(gs_ref, gd_ref, ee_ref, we, att, wa_ref):
    gs = gs_ref[...]
    m = gs + gd_ref[...] + _dot(ee_ref[...], we[...])
    m = jnp.where(m >= 0.0, m, 0.2 * m)
    lg = _dot(m, att[...])
    a = jnp.exp(jnp.minimum(lg, 60.0))
    col = lax.broadcasted_iota(jnp.int32, (_LB, 16), 1)
    a16 = jnp.where(col == 0, a, 0.0)
    wa_ref[...] = jnp.concatenate([gs * a, a16], axis=1)


def _fused(gs, gd, ee, We, att):
    grid = E // _LB
    return pl.pallas_call(
        _fused_body,
        grid=(grid,),
        in_specs=[pl.BlockSpec((_LB, H), lambda i: (i, 0)),
                  pl.BlockSpec((_LB, H), lambda i: (i, 0)),
                  pl.BlockSpec((_LB, H), lambda i: (i, 0)),
                  pl.BlockSpec((H, H), lambda i: (0, 0)),
                  pl.BlockSpec((H, 1), lambda i: (0, 0))],
        out_specs=pl.BlockSpec((_LB, _WA), lambda i: (i, 0)),
        out_shape=jax.ShapeDtypeStruct((E, _WA), jnp.float32),
    )(gs, gd, ee, We, att)


def _norm_body(acc_ref, b_ref, out_ref):
    num = acc_ref[0, :, :H] + acc_ref[1, :, :H]
    den = acc_ref[0, :, H:H + 1] + acc_ref[1, :, H:H + 1]
    out_ref[...] = jnp.maximum(num / (den + 1e-16) + b_ref[...], 0.0)


def _norm(acc, b):
    return pl.pallas_call(
        _norm_body,
        out_shape=jax.ShapeDtypeStruct((N, H), jnp.float32),
    )(acc, b)


_DB = 4000


def _dec_body(hs_ref, hd_ref, w0a, w0b, b0, w1, b1, w2, b2, out_ref):
    h = jnp.maximum(
        _dot(hs_ref[...], w0a[...]) + _dot(hd_ref[...], w0b[...]) + b0[...],
        0.0)
    h = jnp.maximum(_dot(h, w1[...]) + b1[...], 0.0)
    out_ref[...] = _dot(h, w2[...]) + b2[...]


def _decode(hs, hd, p):
    grid = E // _DB
    bspec = pl.BlockSpec((1, H), lambda i: (0, 0))
    return pl.pallas_call(
        _dec_body,
        grid=(grid,),
        in_specs=[pl.BlockSpec((_DB, 2 * H), lambda i: (i, 0)),
                  pl.BlockSpec((_DB, 2 * H), lambda i: (i, 0)),
                  pl.BlockSpec((2 * H, H), lambda i: (0, 0)),
                  pl.BlockSpec((2 * H, H), lambda i: (0, 0)),
                  bspec,
                  pl.BlockSpec((H, H), lambda i: (0, 0)), bspec,
                  pl.BlockSpec((H, 1), lambda i: (0, 0)),
                  pl.BlockSpec((1, 1), lambda i: (0, 0))],
        out_specs=pl.BlockSpec((_DB, 1), lambda i: (i, 0)),
        out_shape=jax.ShapeDtypeStruct((E, 1), jnp.float32),
    )(hs, hd,
      p['dec_w0'][:2 * H], p['dec_w0'][2 * H:], p['dec_b0'].reshape(1, H),
      p['dec_w1'], p['dec_b1'].reshape(1, H),
      p['dec_w2'], p['dec_b2'].reshape(1, 1))


_SC_PARAMS = pltpu.CompilerParams(use_tc_tiling_on_sc=False)


def _gather_pipe(table, idx_hbm, out_hbm, base, nch, CH, idx_v, rows_v, sems):
    pltpu.sync_copy(idx_hbm.at[pl.ds(base, CH)], idx_v[0])
    for i in range(nch):
        b = i % 2
        pltpu.async_copy(table.at[idx_v[b]], rows_v[b], sems[b])
        if i > 0:
            pb = (i - 1) % 2
            pltpu.make_async_copy(table.at[idx_v[pb]], rows_v[pb],
                                  sems[pb]).wait()
            pltpu.sync_copy(rows_v[pb],
                            out_hbm.at[pl.ds(base + (i - 1) * CH, CH)])
        if i + 1 < nch:
            pltpu.sync_copy(idx_hbm.at[pl.ds(base + (i + 1) * CH, CH)],
                            idx_v[(i + 1) % 2])
    lb = (nch - 1) % 2
    pltpu.make_async_copy(table.at[idx_v[lb]], rows_v[lb], sems[lb]).wait()
    pltpu.sync_copy(rows_v[lb], out_hbm.at[pl.ds(base + (nch - 1) * CH, CH)])


def _sc_gather2(table_a, table_b, idx_a, idx_b, D, CH):
    mesh = plsc.VectorSubcoreMesh(core_axis_name="c", subcore_axis_name="s")
    nch = EPW // CH

    @functools.partial(
        pl.kernel, mesh=mesh,
        out_type=(jax.ShapeDtypeStruct((E, D), jnp.float32),
                  jax.ShapeDtypeStruct((E, D), jnp.float32)),
        scratch_types=[pltpu.VMEM((CH,), jnp.int32),
                       pltpu.VMEM((CH,), jnp.int32),
                       pltpu.VMEM((CH, D), jnp.float32),
                       pltpu.VMEM((CH, D), jnp.float32),
                       pltpu.SemaphoreType.DMA,
                       pltpu.SemaphoreType.DMA],
        compiler_params=_SC_PARAMS,
    )
    def k(ta, tb, ia, ib, oa, ob, i0, i1, r0, r1, s0, s1):
        wid = lax.axis_index("c") * NS + lax.axis_index("s")
        base = wid * EPW
        _gather_pipe(ta, ia, oa, base, nch, CH, [i0, i1], [r0, r1], [s0, s1])
        _gather_pipe(tb, ib, ob, base, nch, CH, [i0, i1], [r0, r1], [s0, s1])

    return k(table_a, table_b, idx_a, idx_b)


_SCH = 400
_NRS = N // NS


def _sc_scatter(wa, dst, z80):
    mesh = plsc.VectorSubcoreMesh(core_axis_name="c", subcore_axis_name="s")
    nch = EPW // _SCH

    @functools.partial(
        pl.kernel, mesh=mesh,
        out_type=jax.ShapeDtypeStruct((NC, N, _WA), jnp.float32),
        scratch_types=[pltpu.VMEM((_SCH,), jnp.int32),
                       pltpu.VMEM((_SCH,), jnp.int32),
                       pltpu.VMEM((_SCH, _WA), jnp.float32),
                       pltpu.VMEM((_SCH, _WA), jnp.float32),
                       pltpu.VMEM_SHARED((N, _WA), jnp.float32),
                       pltpu.SemaphoreType.DMA,
                       pltpu.SemaphoreType.DMA],
        compiler_params=_SC_PARAMS,
    )
    def k(wa_hbm, d_hbm, z80_hbm, acc_out, i0, i1, w0, w1, acc_s, s0, s1):
        c = lax.axis_index("c")
        s = lax.axis_index("s")
        rb = s * _NRS

        pltpu.sync_copy(z80_hbm.at[pl.ds(rb, _NRS)],
                        acc_s.at[pl.ds(rb, _NRS)])
        plsc.subcore_barrier()

        base = (c * NS + s) * EPW
        idx_v, w_v = [i0, i1], [w0, w1]
        sems = [s0, s1]

        def load(i):
            b = i % 2
            off = base + i * _SCH
            pltpu.async_copy(d_hbm.at[pl.ds(off, _SCH)], idx_v[b], sems[b])
            pltpu.async_copy(wa_hbm.at[pl.ds(off, _SCH)], w_v[b], sems[b])

        def wait_load(i):
            b = i % 2
            off = base + i * _SCH
            pltpu.make_async_copy(d_hbm.at[pl.ds(off, _SCH)], idx_v[b],
                                  sems[b]).wait()
            pltpu.make_async_copy(wa_hbm.at[pl.ds(off, _SCH)], w_v[b],
                                  sems[b]).wait()

        load(0)
        for i in range(nch):
            b = i % 2
            wait_load(i)
            if i + 1 < nch:
                load(i + 1)
            pltpu.sync_copy(w_v[b], acc_s.at[idx_v[b]], add=True)

        plsc.subcore_barrier()
        pltpu.sync_copy(acc_s.at[pl.ds(rb, _NRS)],
                        acc_out.at[c, pl.ds(rb, _NRS)])

    return k(wa, dst, z80)


def _conv(h, src, dst, ee, Wl, Wr, We, att, b, z80):
    xl, xr = _xlxr(h, Wl, Wr)
    gs, gd = _sc_gather2(xl, xr, src, dst, H, 400)
    wa = _fused(gs, gd, ee, We, att)
    acc = _sc_scatter(wa, dst, z80)
    return _norm(acc, b)


def kernel(x1, edge_index1, edge_attr1, batch1, x2, edge_index2, edge_attr2,
           t_value, params):
    p = params
    src1 = edge_index1[0].astype(jnp.int32)
    dst1 = edge_index1[1].astype(jnp.int32)
    src2 = edge_index2[0].astype(jnp.int32)
    dst2 = edge_index2[1].astype(jnp.int32)
    batch2d = batch1.reshape(N, 1).astype(jnp.int32)

    t_enc = _encode(t_value, batch2d, p)
    s_all = jnp.concatenate([edge_attr1[:, 0:1], edge_attr2], axis=0)
    ee_all = _edge_mlp(s_all, p)
    ee1 = ee_all[:E]
    ee2 = ee_all[E:]

    h1 = jnp.concatenate([t_enc, t_enc], axis=1)
    h2 = h1
    z80 = jnp.zeros((N, _WA), jnp.float32)

    for i in range(L):
        o1 = _conv(h1, src1, dst1, ee1,
                   p['gg_Wl'][i], p['gg_Wr'][i], p['gg_We'][i],
                   p['gg_att'][i].reshape(H, 1), p['gg_b'][i].reshape(1, H),
                   z80)
        o2 = _conv(h2, src2, dst2, ee2,
                   p['gf_Wl'][i], p['gf_Wr'][i], p['gf_We'][i],
                   p['gf_att'][i].reshape(H, 1), p['gf_b'][i].reshape(1, H),
                   z80)
        h1 = jnp.concatenate([o2, o1], axis=1)
        h2 = jnp.concatenate([o1, o2], axis=1)

    hs, hd = _sc_gather2(h1, h1, src1, dst1, 2 * H, 200)
    return _decode(hs, hd, p)


# speedup vs baseline: 6.5683x; 1.0248x over previous
import functools

import jax
import jax.numpy as jnp
from jax import lax
from jax.experimental import pallas as pl
from jax.experimental.pallas import tpu as pltpu
from jax.experimental.pallas import tpu_sc as plsc

H = 64
N = 10000
E = 320000
B = 128
L = 3

NC, NS = 2, 16
NW = NC * NS
EPW = E // NW

_PREC = lax.Precision.DEFAULT


def _dot(a, b):
    return jnp.dot(a, b, precision=_PREC, preferred_element_type=jnp.float32)


def _enc_body(t_ref, batch_ref, w0, b0, w1, b1, w2, b2, out_ref):
    t = t_ref[...]
    h = jnp.maximum(t * w0[...][0:1, :] + b0[...], 0.0)
    h = jnp.maximum(_dot(h, w1[...]) + b1[...], 0.0)
    enc = _dot(h, w2[...]) + b2[...]
    iota = lax.broadcasted_iota(jnp.int32, (N, B), 1)
    oh = (batch_ref[...] == iota).astype(jnp.float32)
    out_ref[...] = _dot(oh, enc)


def _encode(t_value, batch2d, p):
    return pl.pallas_call(
        _enc_body,
        out_shape=jax.ShapeDtypeStruct((N, H), jnp.float32),
    )(t_value, batch2d,
      p['te_w0'], p['te_b0'].reshape(1, H),
      p['te_w1'], p['te_b1'].reshape(1, H),
      p['te_w2'], p['te_b2'].reshape(1, H))


_EEB = 8000


def _ee_body(s_ref, w0, b0, w1, b1, w2, b2, out_ref):
    s = s_ref[...]
    h = jnp.maximum(s * w0[...][0:1, :] + b0[...], 0.0)
    h = jnp.maximum(_dot(h, w1[...]) + b1[...], 0.0)
    out_ref[...] = _dot(h, w2[...]) + b2[...]


def _edge_mlp(s_all, p):
    n = s_all.shape[0]
    grid = n // _EEB
    wspec = pl.BlockSpec((1, H), lambda i: (0, 0))
    return pl.pallas_call(
        _ee_body,
        grid=(grid,),
        in_specs=[pl.BlockSpec((_EEB, 1), lambda i: (i, 0)),
                  wspec, wspec, pl.BlockSpec((H, H), lambda i: (0, 0)), wspec,
                  pl.BlockSpec((H, H), lambda i: (0, 0)), wspec],
        out_specs=pl.BlockSpec((_EEB, H), lambda i: (i, 0)),
        out_shape=jax.ShapeDtypeStruct((n, H), jnp.float32),
    )(s_all,
      p['ee_w0'], p['ee_b0'].reshape(1, H),
      p['ee_w1'], p['ee_b1'].reshape(1, H),
      p['ee_w2'], p['ee_b2'].reshape(1, H))


def _xlxr_body(h_ref, wl, wr, xl_ref, xr_ref):
    h = h_ref[...]
    xl_ref[...] = _dot(h, wl[...])
    xr_ref[...] = _dot(h, wr[...])


def _xlxr(h, Wl, Wr):
    return pl.pallas_call(
        _xlxr_body,
        out_shape=(jax.ShapeDtypeStruct((N, H), jnp.float32),
                   jax.ShapeDtypeStruct((N, H), jnp.float32)),
    )(h, Wl, Wr)


_LB = 8000
_WA = H + 16


def _fused_body(gs_ref, gd_ref, ee_ref, we, att, wa_ref):
    gs = gs_ref[...]
    m = gs + gd_ref[...] + _dot(ee_ref[...], we[...])
    m = jnp.where(m >= 0.0, m, 0.2 * m)
    lg = _dot(m, att[...])
    a = jnp.exp(jnp.minimum(lg, 60.0))
    col = lax.broadcasted_iota(jnp.int32, (_LB, 16), 1)
    a16 = jnp.where(col == 0, a, 0.0)
    wa_ref[...] = jnp.concatenate([gs * a, a16], axis=1)


def _fused(gs, gd, ee, We, att):
    grid = E // _LB
    return pl.pallas_call(
        _fused_body,
        grid=(grid,),
        in_specs=[pl.BlockSpec((_LB, H), lambda i: (i, 0)),
                  pl.BlockSpec((_LB, H), lambda i: (i, 0)),
                  pl.BlockSpec((_LB, H), lambda i: (i, 0)),
                  pl.BlockSpec((H, H), lambda i: (0, 0)),
                  pl.BlockSpec((H, 1), lambda i: (0, 0))],
        out_specs=pl.BlockSpec((_LB, _WA), lambda i: (i, 0)),
        out_shape=jax.ShapeDtypeStruct((E, _WA), jnp.float32),
    )(gs, gd, ee, We, att)


def _norm_body(acc_ref, b_ref, out_ref):
    num = acc_ref[0, :, :H] + acc_ref[1, :, :H]
    den = acc_ref[0, :, H:H + 1] + acc_ref[1, :, H:H + 1]
    out_ref[...] = jnp.maximum(num / (den + 1e-16) + b_ref[...], 0.0)


def _norm(acc, b):
    return pl.pallas_call(
        _norm_body,
        out_shape=jax.ShapeDtypeStruct((N, H), jnp.float32),
    )(acc, b)


_DB = 4000


def _dec_body(hs_ref, hd_ref, w0a, w0b, b0, w1, b1, w2, b2, out_ref):
    h = jnp.maximum(
        _dot(hs_ref[...], w0a[...]) + _dot(hd_ref[...], w0b[...]) + b0[...],
        0.0)
    h = jnp.maximum(_dot(h, w1[...]) + b1[...], 0.0)
    out_ref[...] = _dot(h, w2[...]) + b2[...]


def _decode(hs, hd, p):
    grid = E // _DB
    bspec = pl.BlockSpec((1, H), lambda i: (0, 0))
    return pl.pallas_call(
        _dec_body,
        grid=(grid,),
        in_specs=[pl.BlockSpec((_DB, 2 * H), lambda i: (i, 0)),
                  pl.BlockSpec((_DB, 2 * H), lambda i: (i, 0)),
                  pl.BlockSpec((2 * H, H), lambda i: (0, 0)),
                  pl.BlockSpec((2 * H, H), lambda i: (0, 0)),
                  bspec,
                  pl.BlockSpec((H, H), lambda i: (0, 0)), bspec,
                  pl.BlockSpec((H, 1), lambda i: (0, 0)),
                  pl.BlockSpec((1, 1), lambda i: (0, 0))],
        out_specs=pl.BlockSpec((_DB, 1), lambda i: (i, 0)),
        out_shape=jax.ShapeDtypeStruct((E, 1), jnp.float32),
    )(hs, hd,
      p['dec_w0'][:2 * H], p['dec_w0'][2 * H:], p['dec_b0'].reshape(1, H),
      p['dec_w1'], p['dec_b1'].reshape(1, H),
      p['dec_w2'], p['dec_b2'].reshape(1, 1))


_SC_PARAMS = pltpu.CompilerParams(use_tc_tiling_on_sc=False)


def _gather_pipe(table, idx_hbm, out_hbm, base, nch, CH, idx_v, rows_v, sems):
    pltpu.sync_copy(idx_hbm.at[pl.ds(base, CH)], idx_v[0])
    for i in range(nch):
        b = i % 2
        pltpu.async_copy(table.at[idx_v[b]], rows_v[b], sems[b])
        if i > 0:
            pb = (i - 1) % 2
            pltpu.make_async_copy(table.at[idx_v[pb]], rows_v[pb],
                                  sems[pb]).wait()
            pltpu.sync_copy(rows_v[pb],
                            out_hbm.at[pl.ds(base + (i - 1) * CH, CH)])
        if i + 1 < nch:
            pltpu.sync_copy(idx_hbm.at[pl.ds(base + (i + 1) * CH, CH)],
                            idx_v[(i + 1) % 2])
    lb = (nch - 1) % 2
    pltpu.make_async_copy(table.at[idx_v[lb]], rows_v[lb], sems[lb]).wait()
    pltpu.sync_copy(rows_v[lb], out_hbm.at[pl.ds(base + (nch - 1) * CH, CH)])


def _sc_gather2(table_a, table_b, idx_a, idx_b, D, CH, use_spmem=True):
    mesh = plsc.VectorSubcoreMesh(core_axis_name="c", subcore_axis_name="s")
    nch = EPW // CH
    scratch = [pltpu.VMEM((CH,), jnp.int32),
               pltpu.VMEM((CH,), jnp.int32),
               pltpu.VMEM((CH, D), jnp.float32),
               pltpu.VMEM((CH, D), jnp.float32)]
    if use_spmem:
        scratch.append(pltpu.VMEM_SHARED((N, D), jnp.float32))
    scratch += [pltpu.SemaphoreType.DMA, pltpu.SemaphoreType.DMA]

    @functools.partial(
        pl.kernel, mesh=mesh,
        out_type=(jax.ShapeDtypeStruct((E, D), jnp.float32),
                  jax.ShapeDtypeStruct((E, D), jnp.float32)),
        scratch_types=scratch,
        compiler_params=_SC_PARAMS,
    )
    def k(ta, tb, ia, ib, oa, ob, i0, i1, r0, r1, *rest):
        if use_spmem:
            tab_s, s0, s1 = rest
        else:
            s0, s1 = rest
        s = lax.axis_index("s")
        base = (lax.axis_index("c") * NS + s) * EPW
        if use_spmem:
            rb = s * _NRS
            pltpu.sync_copy(ta.at[pl.ds(rb, _NRS)], tab_s.at[pl.ds(rb, _NRS)])
            plsc.subcore_barrier()
            src_a = tab_s
        else:
            src_a = ta
        _gather_pipe(src_a, ia, oa, base, nch, CH,
                     [i0, i1], [r0, r1], [s0, s1])
        _gather_pipe(tb, ib, ob, base, nch, CH,
                     [i0, i1], [r0, r1], [s0, s1])

    return k(table_a, table_b, idx_a, idx_b)


_SCH = 400
_NRS = N // NS


def _sc_scatter(wa, dst, z80):
    mesh = plsc.VectorSubcoreMesh(core_axis_name="c", subcore_axis_name="s")
    nch = EPW // _SCH

    @functools.partial(
        pl.kernel, mesh=mesh,
        out_type=jax.ShapeDtypeStruct((NC, N, _WA), jnp.float32),
        scratch_types=[pltpu.VMEM((_SCH,), jnp.int32),
                       pltpu.VMEM((_SCH,), jnp.int32),
                       pltpu.VMEM((_SCH, _WA), jnp.float32),
                       pltpu.VMEM((_SCH, _WA), jnp.float32),
                       pltpu.VMEM_SHARED((N, _WA), jnp.float32),
                       pltpu.SemaphoreType.DMA,
                       pltpu.SemaphoreType.DMA],
        compiler_params=_SC_PARAMS,
    )
    def k(wa_hbm, d_hbm, z80_hbm, acc_out, i0, i1, w0, w1, acc_s, s0, s1):
        c = lax.axis_index("c")
        s = lax.axis_index("s")
        rb = s * _NRS

        pltpu.sync_copy(z80_hbm.at[pl.ds(rb, _NRS)],
                        acc_s.at[pl.ds(rb, _NRS)])
        plsc.subcore_barrier()

        base = (c * NS + s) * EPW
        idx_v, w_v = [i0, i1], [w0, w1]
        sems = [s0, s1]

        def load(i):
            b = i % 2
            off = base + i * _SCH
            pltpu.async_copy(d_hbm.at[pl.ds(off, _SCH)], idx_v[b], sems[b])
            pltpu.async_copy(wa_hbm.at[pl.ds(off, _SCH)], w_v[b], sems[b])

        def wait_load(i):
            b = i % 2
            off = base + i * _SCH
            pltpu.make_async_copy(d_hbm.at[pl.ds(off, _SCH)], idx_v[b],
                                  sems[b]).wait()
            pltpu.make_async_copy(wa_hbm.at[pl.ds(off, _SCH)], w_v[b],
                                  sems[b]).wait()

        load(0)
        for i in range(nch):
            b = i % 2
            wait_load(i)
            if i + 1 < nch:
                load(i + 1)
            pltpu.sync_copy(w_v[b], acc_s.at[idx_v[b]], add=True)

        plsc.subcore_barrier()
        pltpu.sync_copy(acc_s.at[pl.ds(rb, _NRS)],
                        acc_out.at[c, pl.ds(rb, _NRS)])

    return k(wa, dst, z80)


def _conv(h, src, dst, ee, Wl, Wr, We, att, b, z80):
    xl, xr = _xlxr(h, Wl, Wr)
    gs, gd = _sc_gather2(xl, xr, src, dst, H, 400)
    wa = _fused(gs, gd, ee, We, att)
    acc = _sc_scatter(wa, dst, z80)
    return _norm(acc, b)


def kernel(x1, edge_index1, edge_attr1, batch1, x2, edge_index2, edge_attr2,
           t_value, params):
    p = params
    src1 = edge_index1[0].astype(jnp.int32)
    dst1 = edge_index1[1].astype(jnp.int32)
    src2 = edge_index2[0].astype(jnp.int32)
    dst2 = edge_index2[1].astype(jnp.int32)
    batch2d = batch1.reshape(N, 1).astype(jnp.int32)

    t_enc = _encode(t_value, batch2d, p)
    s_all = jnp.concatenate([edge_attr1[:, 0:1], edge_attr2], axis=0)
    ee_all = _edge_mlp(s_all, p)
    ee1 = ee_all[:E]
    ee2 = ee_all[E:]

    h1 = jnp.concatenate([t_enc, t_enc], axis=1)
    h2 = h1
    z80 = jnp.zeros((N, _WA), jnp.float32)

    for i in range(L):
        o1 = _conv(h1, src1, dst1, ee1,
                   p['gg_Wl'][i], p['gg_Wr'][i], p['gg_We'][i],
                   p['gg_att'][i].reshape(H, 1), p['gg_b'][i].reshape(1, H),
                   z80)
        o2 = _conv(h2, src2, dst2, ee2,
                   p['gf_Wl'][i], p['gf_Wr'][i], p['gf_We'][i],
                   p['gf_att'][i].reshape(H, 1), p['gf_b'][i].reshape(1, H),
                   z80)
        h1 = jnp.concatenate([o2, o1], axis=1)
        h2 = jnp.concatenate([o1, o2], axis=1)

    hs, hd = _sc_gather2(h1, h1, src1, dst1, 2 * H, 200, use_spmem=False)
    return _decode(hs, hd, p)


# speedup vs baseline: 6.6013x; 1.0050x over previous
import functools

import jax
import jax.numpy as jnp
from jax import lax
from jax.experimental import pallas as pl
from jax.experimental.pallas import tpu as pltpu
from jax.experimental.pallas import tpu_sc as plsc

H = 64
N = 10000
E = 320000
B = 128
L = 3

NC, NS = 2, 16
NW = NC * NS
EPW = E // NW

_PREC = lax.Precision.DEFAULT


def _dot(a, b):
    return jnp.dot(a, b, precision=_PREC, preferred_element_type=jnp.float32)


def _enc_body(t_ref, batch_ref, w0, b0, w1, b1, w2, b2, out_ref):
    t = t_ref[...]
    h = jnp.maximum(t * w0[...][0:1, :] + b0[...], 0.0)
    h = jnp.maximum(_dot(h, w1[...]) + b1[...], 0.0)
    enc = _dot(h, w2[...]) + b2[...]
    iota = lax.broadcasted_iota(jnp.int32, (N, B), 1)
    oh = (batch_ref[...] == iota).astype(jnp.float32)
    out_ref[...] = _dot(oh, enc)


def _encode(t_value, batch2d, p):
    return pl.pallas_call(
        _enc_body,
        out_shape=jax.ShapeDtypeStruct((N, H), jnp.float32),
    )(t_value, batch2d,
      p['te_w0'], p['te_b0'].reshape(1, H),
      p['te_w1'], p['te_b1'].reshape(1, H),
      p['te_w2'], p['te_b2'].reshape(1, H))


_EEB = 8000


def _ee_body(s_ref, w0, b0, w1, b1, w2, b2, out_ref):
    s = s_ref[...]
    h = jnp.maximum(s * w0[...][0:1, :] + b0[...], 0.0)
    h = jnp.maximum(_dot(h, w1[...]) + b1[...], 0.0)
    out_ref[...] = _dot(h, w2[...]) + b2[...]


def _edge_mlp(s_all, p):
    n = s_all.shape[0]
    grid = n // _EEB
    wspec = pl.BlockSpec((1, H), lambda i: (0, 0))
    return pl.pallas_call(
        _ee_body,
        grid=(grid,),
        in_specs=[pl.BlockSpec((_EEB, 1), lambda i: (i, 0)),
                  wspec, wspec, pl.BlockSpec((H, H), lambda i: (0, 0)), wspec,
                  pl.BlockSpec((H, H), lambda i: (0, 0)), wspec],
        out_specs=pl.BlockSpec((_EEB, H), lambda i: (i, 0)),
        out_shape=jax.ShapeDtypeStruct((n, H), jnp.float32),
    )(s_all,
      p['ee_w0'], p['ee_b0'].reshape(1, H),
      p['ee_w1'], p['ee_b1'].reshape(1, H),
      p['ee_w2'], p['ee_b2'].reshape(1, H))


def _xlxr_body(h_ref, wl, wr, xl_ref, xr_ref):
    h = h_ref[...]
    xl_ref[...] = _dot(h, wl[...])
    xr_ref[...] = _dot(h, wr[...])


def _xlxr(h, Wl, Wr):
    return pl.pallas_call(
        _xlxr_body,
        out_shape=(jax.ShapeDtypeStruct((N, H), jnp.float32),
                   jax.ShapeDtypeStruct((N, H), jnp.float32)),
    )(h, Wl, Wr)


_LB = 8000
_WA = H + 16


def _fused_body(gs_ref, gd_ref, ee_ref, we, att, wa_ref):
    gs = gs_ref[...]
    m = gs + gd_ref[...] + _dot(ee_ref[...], we[...])
    m = jnp.where(m >= 0.0, m, 0.2 * m)
    lg = _dot(m, att[...])
    a = jnp.exp(jnp.minimum(lg, 60.0))
    col = lax.broadcasted_iota(jnp.int32, (_LB, 16), 1)
    a16 = jnp.where(col == 0, a, 0.0)
    wa_ref[...] = jnp.concatenate([gs * a, a16], axis=1)


def _fused(gs, gd, ee, We, att):
    grid = E // _LB
    return pl.pallas_call(
        _fused_body,
        grid=(grid,),
        in_specs=[pl.BlockSpec((_LB, H), lambda i: (i, 0)),
                  pl.BlockSpec((_LB, H), lambda i: (i, 0)),
                  pl.BlockSpec((_LB, H), lambda i: (i, 0)),
                  pl.BlockSpec((H, H), lambda i: (0, 0)),
                  pl.BlockSpec((H, 1), lambda i: (0, 0))],
        out_specs=pl.BlockSpec((_LB, _WA), lambda i: (i, 0)),
        out_shape=jax.ShapeDtypeStruct((E, _WA), jnp.float32),
    )(gs, gd, ee, We, att)


def _n2p_body(acc1_ref, acc2_ref, b1, b2,
              wl1t, wl1b, wr1t, wr1b, wl2t, wl2b, wr2t, wr2b,
              xl1_ref, xr1_ref, xl2_ref, xr2_ref):
    num1 = acc1_ref[0, :, :H] + acc1_ref[1, :, :H]
    den1 = acc1_ref[0, :, H:H + 1] + acc1_ref[1, :, H:H + 1]
    o1 = jnp.maximum(num1 / (den1 + 1e-16) + b1[...], 0.0)
    num2 = acc2_ref[0, :, :H] + acc2_ref[1, :, :H]
    den2 = acc2_ref[0, :, H:H + 1] + acc2_ref[1, :, H:H + 1]
    o2 = jnp.maximum(num2 / (den2 + 1e-16) + b2[...], 0.0)
    xl1_ref[...] = _dot(o2, wl1t[...]) + _dot(o1, wl1b[...])
    xr1_ref[...] = _dot(o2, wr1t[...]) + _dot(o1, wr1b[...])
    xl2_ref[...] = _dot(o1, wl2t[...]) + _dot(o2, wl2b[...])
    xr2_ref[...] = _dot(o1, wr2t[...]) + _dot(o2, wr2b[...])


def _norm2proj(acc1, acc2, b1, b2, Wl1, Wr1, Wl2, Wr2):
    out = jax.ShapeDtypeStruct((N, H), jnp.float32)
    return pl.pallas_call(
        _n2p_body,
        out_shape=(out, out, out, out),
    )(acc1, acc2, b1, b2,
      Wl1[:H], Wl1[H:], Wr1[:H], Wr1[H:],
      Wl2[:H], Wl2[H:], Wr2[:H], Wr2[H:])


def _norm_body(acc_ref, b_ref, out_ref):
    num = acc_ref[0, :, :H] + acc_ref[1, :, :H]
    den = acc_ref[0, :, H:H + 1] + acc_ref[1, :, H:H + 1]
    out_ref[...] = jnp.maximum(num / (den + 1e-16) + b_ref[...], 0.0)


def _norm(acc, b):
    return pl.pallas_call(
        _norm_body,
        out_shape=jax.ShapeDtypeStruct((N, H), jnp.float32),
    )(acc, b)


_DB = 4000


def _dec_body(hs_ref, hd_ref, w0a, w0b, b0, w1, b1, w2, b2, out_ref):
    h = jnp.maximum(
        _dot(hs_ref[...], w0a[...]) + _dot(hd_ref[...], w0b[...]) + b0[...],
        0.0)
    h = jnp.maximum(_dot(h, w1[...]) + b1[...], 0.0)
    out_ref[...] = _dot(h, w2[...]) + b2[...]


def _decode(hs, hd, p):
    grid = E // _DB
    bspec = pl.BlockSpec((1, H), lambda i: (0, 0))
    return pl.pallas_call(
        _dec_body,
        grid=(grid,),
        in_specs=[pl.BlockSpec((_DB, 2 * H), lambda i: (i, 0)),
                  pl.BlockSpec((_DB, 2 * H), lambda i: (i, 0)),
                  pl.BlockSpec((2 * H, H), lambda i: (0, 0)),
                  pl.BlockSpec((2 * H, H), lambda i: (0, 0)),
                  bspec,
                  pl.BlockSpec((H, H), lambda i: (0, 0)), bspec,
                  pl.BlockSpec((H, 1), lambda i: (0, 0)),
                  pl.BlockSpec((1, 1), lambda i: (0, 0))],
        out_specs=pl.BlockSpec((_DB, 1), lambda i: (i, 0)),
        out_shape=jax.ShapeDtypeStruct((E, 1), jnp.float32),
    )(hs, hd,
      p['dec_w0'][:2 * H], p['dec_w0'][2 * H:], p['dec_b0'].reshape(1, H),
      p['dec_w1'], p['dec_b1'].reshape(1, H),
      p['dec_w2'], p['dec_b2'].reshape(1, 1))


_SC_PARAMS = pltpu.CompilerParams(use_tc_tiling_on_sc=False)


def _gather_pipe(table, idx_hbm, out_hbm, base, nch, CH, idx_v, rows_v, sems):
    pltpu.sync_copy(idx_hbm.at[pl.ds(base, CH)], idx_v[0])
    for i in range(nch):
        b = i % 2
        pltpu.async_copy(table.at[idx_v[b]], rows_v[b], sems[b])
        if i > 0:
            pb = (i - 1) % 2
            pltpu.make_async_copy(table.at[idx_v[pb]], rows_v[pb],
                                  sems[pb]).wait()
            pltpu.sync_copy(rows_v[pb],
                            out_hbm.at[pl.ds(base + (i - 1) * CH, CH)])
        if i + 1 < nch:
            pltpu.sync_copy(idx_hbm.at[pl.ds(base + (i + 1) * CH, CH)],
                            idx_v[(i + 1) % 2])
    lb = (nch - 1) % 2
    pltpu.make_async_copy(table.at[idx_v[lb]], rows_v[lb], sems[lb]).wait()
    pltpu.sync_copy(rows_v[lb], out_hbm.at[pl.ds(base + (nch - 1) * CH, CH)])


def _sc_gather2(table_a, table_b, idx_a, idx_b, D, CH, use_spmem=True):
    mesh = plsc.VectorSubcoreMesh(core_axis_name="c", subcore_axis_name="s")
    nch = EPW // CH
    scratch = [pltpu.VMEM((CH,), jnp.int32),
               pltpu.VMEM((CH,), jnp.int32),
               pltpu.VMEM((CH, D), jnp.float32),
               pltpu.VMEM((CH, D), jnp.float32)]
    if use_spmem:
        scratch.append(pltpu.VMEM_SHARED((N, D), jnp.float32))
    scratch += [pltpu.SemaphoreType.DMA, pltpu.SemaphoreType.DMA]

    @functools.partial(
        pl.kernel, mesh=mesh,
        out_type=(jax.ShapeDtypeStruct((E, D), jnp.float32),
                  jax.ShapeDtypeStruct((E, D), jnp.float32)),
        scratch_types=scratch,
        compiler_params=_SC_PARAMS,
    )
    def k(ta, tb, ia, ib, oa, ob, i0, i1, r0, r1, *rest):
        if use_spmem:
            tab_s, s0, s1 = rest
        else:
            s0, s1 = rest
        s = lax.axis_index("s")
        base = (lax.axis_index("c") * NS + s) * EPW
        if use_spmem:
            rb = s * _NRS
            pltpu.sync_copy(ta.at[pl.ds(rb, _NRS)], tab_s.at[pl.ds(rb, _NRS)])
            plsc.subcore_barrier()
            src_a = tab_s
        else:
            src_a = ta
        _gather_pipe(src_a, ia, oa, base, nch, CH,
                     [i0, i1], [r0, r1], [s0, s1])
        _gather_pipe(tb, ib, ob, base, nch, CH,
                     [i0, i1], [r0, r1], [s0, s1])

    return k(table_a, table_b, idx_a, idx_b)


_SCH = 400
_NRS = N // NS


def _sc_scatter(wa, dst, z80):
    mesh = plsc.VectorSubcoreMesh(core_axis_name="c", subcore_axis_name="s")
    nch = EPW // _SCH

    @functools.partial(
        pl.kernel, mesh=mesh,
        out_type=jax.ShapeDtypeStruct((NC, N, _WA), jnp.float32),
        scratch_types=[pltpu.VMEM((_SCH,), jnp.int32),
                       pltpu.VMEM((_SCH,), jnp.int32),
                       pltpu.VMEM((_SCH, _WA), jnp.float32),
                       pltpu.VMEM((_SCH, _WA), jnp.float32),
                       pltpu.VMEM_SHARED((N, _WA), jnp.float32),
                       pltpu.SemaphoreType.DMA,
                       pltpu.SemaphoreType.DMA],
        compiler_params=_SC_PARAMS,
    )
    def k(wa_hbm, d_hbm, z80_hbm, acc_out, i0, i1, w0, w1, acc_s, s0, s1):
        c = lax.axis_index("c")
        s = lax.axis_index("s")
        rb = s * _NRS

        pltpu.sync_copy(z80_hbm.at[pl.ds(rb, _NRS)],
                        acc_s.at[pl.ds(rb, _NRS)])
        plsc.subcore_barrier()

        base = (c * NS + s) * EPW
        idx_v, w_v = [i0, i1], [w0, w1]
        sems = [s0, s1]

        def load(i):
            b = i % 2
            off = base + i * _SCH
            pltpu.async_copy(d_hbm.at[pl.ds(off, _SCH)], idx_v[b], sems[b])
            pltpu.async_copy(wa_hbm.at[pl.ds(off, _SCH)], w_v[b], sems[b])

        def wait_load(i):
            b = i % 2
            off = base + i * _SCH
            pltpu.make_async_copy(d_hbm.at[pl.ds(off, _SCH)], idx_v[b],
                                  sems[b]).wait()
            pltpu.make_async_copy(wa_hbm.at[pl.ds(off, _SCH)], w_v[b],
                                  sems[b]).wait()

        load(0)
        for i in range(nch):
            b = i % 2
            wait_load(i)
            if i + 1 < nch:
                load(i + 1)
            pltpu.sync_copy(w_v[b], acc_s.at[idx_v[b]], add=True)

        plsc.subcore_barrier()
        pltpu.sync_copy(acc_s.at[pl.ds(rb, _NRS)],
                        acc_out.at[c, pl.ds(rb, _NRS)])

    return k(wa, dst, z80)


def _edge_pass(xl, xr, src, dst, ee, We, att, z80):
    gs, gd = _sc_gather2(xl, xr, src, dst, H, 400)
    wa = _fused(gs, gd, ee, We, att)
    return _sc_scatter(wa, dst, z80)


def kernel(x1, edge_index1, edge_attr1, batch1, x2, edge_index2, edge_attr2,
           t_value, params):
    p = params
    src1 = edge_index1[0].astype(jnp.int32)
    dst1 = edge_index1[1].astype(jnp.int32)
    src2 = edge_index2[0].astype(jnp.int32)
    dst2 = edge_index2[1].astype(jnp.int32)
    batch2d = batch1.reshape(N, 1).astype(jnp.int32)

    t_enc = _encode(t_value, batch2d, p)
    s_all = jnp.concatenate([edge_attr1[:, 0:1], edge_attr2], axis=0)
    ee_all = _edge_mlp(s_all, p)
    ee1 = ee_all[:E]
    ee2 = ee_all[E:]

    h0 = jnp.concatenate([t_enc, t_enc], axis=1)
    z80 = jnp.zeros((N, _WA), jnp.float32)

    xl1, xr1 = _xlxr(h0, p['gg_Wl'][0], p['gg_Wr'][0])
    xl2, xr2 = _xlxr(h0, p['gf_Wl'][0], p['gf_Wr'][0])
    for i in range(L):
        acc1 = _edge_pass(xl1, xr1, src1, dst1, ee1, p['gg_We'][i],
                          p['gg_att'][i].reshape(H, 1), z80)
        acc2 = _edge_pass(xl2, xr2, src2, dst2, ee2, p['gf_We'][i],
                          p['gf_att'][i].reshape(H, 1), z80)
        if i + 1 < L:
            xl1, xr1, xl2, xr2 = _norm2proj(
                acc1, acc2,
                p['gg_b'][i].reshape(1, H), p['gf_b'][i].reshape(1, H),
                p['gg_Wl'][i + 1], p['gg_Wr'][i + 1],
                p['gf_Wl'][i + 1], p['gf_Wr'][i + 1])
        else:
            o1 = _norm(acc1, p['gg_b'][i].reshape(1, H))
            o2 = _norm(acc2, p['gf_b'][i].reshape(1, H))
            h1 = jnp.concatenate([o2, o1], axis=1)

    hs, hd = _sc_gather2(h1, h1, src1, dst1, 2 * H, 200, use_spmem=False)
    return _decode(hs, hd, p)
